# baseline (speedup 1.0000x reference)
import functools

import jax
import jax.numpy as jnp
from jax import lax
from jax.experimental import pallas as pl
from jax.experimental.pallas import tpu as pltpu
from jax.experimental.pallas import tpu_sc as plsc

BATCH = 4096
SEQ = 200
EMBED_DIM = 64
HIDDEN = 256
NUM_CLASSES = 50

_NC = 2
_NS = 16
_NW = _NC * _NS
_BPW = BATCH // _NW
_IPW = _BPW * SEQ
_G0, _G1 = 104, 96


def _pool_sc(x_flat, emb):
    mesh = plsc.VectorSubcoreMesh(core_axis_name="c", subcore_axis_name="s")

    @functools.partial(
        pl.kernel,
        mesh=mesh,
        compiler_params=pltpu.CompilerParams(use_tc_tiling_on_sc=False),
        out_type=jax.ShapeDtypeStruct((BATCH, EMBED_DIM), jnp.float32),
        scratch_types=[
            pltpu.VMEM((_IPW,), jnp.int32),
            pltpu.VMEM((SEQ, EMBED_DIM), jnp.float32),
            pltpu.VMEM((_BPW, EMBED_DIM), jnp.float32),
            pltpu.SemaphoreType.DMA,
        ],
    )
    def pool(x_hbm, emb_hbm, out_hbm, idx_v, rows_v, out_v, sem):
        wid = lax.axis_index("s") * _NC + lax.axis_index("c")
        ibase = pl.multiple_of(wid * _IPW, 8)
        pltpu.sync_copy(x_hbm.at[pl.ds(ibase, _IPW)], idx_v)

        def elem(i, _):
            off = pl.multiple_of(i * SEQ, 8)
            cp0 = pltpu.async_copy(
                emb_hbm.at[idx_v.at[pl.ds(off, _G0)]],
                rows_v.at[pl.ds(0, _G0)], sem)
            cp1 = pltpu.async_copy(
                emb_hbm.at[idx_v.at[pl.ds(off + _G0, _G1)]],
                rows_v.at[pl.ds(_G0, _G1)], sem)
            cp0.wait()
            cp1.wait()

            def accum(s, accs):
                return tuple(
                    accs[j] + rows_v[s, pl.ds(j * 16, 16)] for j in range(4))

            accs = lax.fori_loop(
                0, SEQ, accum,
                tuple(jnp.zeros((16,), jnp.float32) for _ in range(4)))
            scale = jnp.float32(1.0 / SEQ)
            for j in range(4):
                out_v[i, pl.ds(j * 16, 16)] = accs[j] * scale
            return 0

        lax.fori_loop(0, _BPW, elem, 0)
        pltpu.sync_copy(out_v, out_hbm.at[pl.ds(wid * _BPW, _BPW)])

    return pool(x_flat, emb)


def _mlp_body(p_ref, w1_ref, b1_ref, w2_ref, b2_ref, o_ref):
    h = jnp.dot(p_ref[...], w1_ref[...], preferred_element_type=jnp.float32)
    h = jnp.maximum(h + b1_ref[...], 0.0)
    o_ref[...] = (
        jnp.dot(h, w2_ref[...], preferred_element_type=jnp.float32)
        + b2_ref[...])


def _mlp_tc(pooled, W1, b1, W2, b2):
    bb = 512
    grid = (BATCH // bb,)
    return pl.pallas_call(
        _mlp_body,
        grid=grid,
        in_specs=[
            pl.BlockSpec((bb, EMBED_DIM), lambda i: (i, 0)),
            pl.BlockSpec((EMBED_DIM, HIDDEN), lambda i: (0, 0)),
            pl.BlockSpec((1, HIDDEN), lambda i: (0, 0)),
            pl.BlockSpec((HIDDEN, NUM_CLASSES), lambda i: (0, 0)),
            pl.BlockSpec((1, NUM_CLASSES), lambda i: (0, 0)),
        ],
        out_specs=pl.BlockSpec((bb, NUM_CLASSES), lambda i: (i, 0)),
        out_shape=jax.ShapeDtypeStruct((BATCH, NUM_CLASSES), jnp.float32),
    )(pooled, W1, b1, W2, b2)


def kernel(x, emb, W1, b1, W2, b2):
    pooled = _pool_sc(x.reshape(-1), emb)
    return _mlp_tc(pooled, W1, b1.reshape(1, HIDDEN), W2,
                   b2.reshape(1, NUM_CLASSES))
